# baseline (speedup 1.0000x reference)
import functools

import jax
import jax.numpy as jnp
from jax.experimental import pallas as pl
from jax.experimental.pallas import tpu as pltpu

_R = 16384
_D = 100
_BI = 16384
_G = _R // _BI
_BA = _BI // 128


def _body(bias_ref, xt_ref, o_ref):
    t = jnp.tanh(bias_ref[...])
    xb = xt_ref[...]
    t0, t1, t2 = t[0, 0], t[1, 0], t[2, 0]
    sel = jnp.where(xb == 0, t0, jnp.where(xb == 1, t1, t2))
    o_ref[...] = sel.reshape(_D, _BA, 128)


@jax.jit
def kernel(x, data_bias):
    xt = x.astype(jnp.int32).T
    res = pl.pallas_call(
        _body,
        grid=(_G,),
        in_specs=[
            pl.BlockSpec((3, 1), lambda i: (0, 0)),
            pl.BlockSpec((_D, _BI), lambda i: (0, i)),
        ],
        out_specs=pl.BlockSpec((_D, _BA, 128), lambda i: (0, i, 0)),
        out_shape=jax.ShapeDtypeStruct((_D, _R // 128, 128), jnp.float32),
    )(data_bias, xt)
    return jnp.transpose(res, (1, 2, 0)).reshape(_R, _D, 1)


# speedup vs baseline: 1.2031x; 1.2031x over previous
import functools

import jax
import jax.numpy as jnp
from jax.experimental import pallas as pl
from jax.experimental.pallas import tpu as pltpu

_R = 16384
_D = 100
_BI = 8192
_G = _R // _BI
_BA = _BI // 128


def _body(bias_ref, xt_ref, o_ref):
    xb = xt_ref[...]
    t0 = jnp.tanh(bias_ref[0, 0])
    t1 = jnp.tanh(bias_ref[1, 0])
    t2 = jnp.tanh(bias_ref[2, 0])
    sel = jnp.where(xb == 0, t0, jnp.where(xb == 1, t1, t2))
    o_ref[...] = sel.reshape(_D, _BA, 128)


@jax.jit
def kernel(x, data_bias):
    xt = x.astype(jnp.int32).T
    res = pl.pallas_call(
        _body,
        grid=(_G,),
        in_specs=[
            pl.BlockSpec(memory_space=pltpu.SMEM),
            pl.BlockSpec((_D, _BI), lambda i: (0, i)),
        ],
        out_specs=pl.BlockSpec((_D, _BA, 128), lambda i: (0, i, 0)),
        out_shape=jax.ShapeDtypeStruct((_D, _R // 128, 128), jnp.float32),
    )(data_bias, xt)
    return jnp.transpose(res, (1, 2, 0)).reshape(_R, _D, 1)


# speedup vs baseline: 1.4267x; 1.1858x over previous
import functools

import jax
import jax.numpy as jnp
from jax.experimental import pallas as pl
from jax.experimental.pallas import tpu as pltpu

_R = 16384
_D = 100
_BI = 8192
_G = _R // _BI
_BA = _BI // 128


def _body(bias_ref, xt_ref, o_ref):
    xb = xt_ref[...]
    t0 = jnp.tanh(bias_ref[0])
    t1 = jnp.tanh(bias_ref[1])
    t2 = jnp.tanh(bias_ref[2])
    sel = jnp.where(xb == 0, t0, jnp.where(xb == 1, t1, t2))
    o_ref[...] = sel.reshape(_D, _BA, 128)


@jax.jit
def kernel(x, data_bias):
    xt = x.astype(jnp.int32).T
    res = pl.pallas_call(
        _body,
        grid=(_G,),
        in_specs=[
            pl.BlockSpec(memory_space=pltpu.SMEM),
            pl.BlockSpec((_D, _BI), lambda i: (0, i)),
        ],
        out_specs=pl.BlockSpec((_D, _BA, 128), lambda i: (0, i, 0)),
        out_shape=jax.ShapeDtypeStruct((_D, _R // 128, 128), jnp.float32),
    )(data_bias.reshape(-1), xt)
    return jnp.transpose(res, (1, 2, 0)).reshape(_R, _D, 1)
